# Initial kernel scaffold; baseline (speedup 1.0000x reference)
import jax
import jax.numpy as jnp
from jax.experimental import pallas as pl


def kernel(x_in, codebook):
    raise NotImplementedError("write your pallas kernel here")


# speedup vs baseline: 1.5949x; 1.5949x over previous
import functools

import jax
import jax.numpy as jnp
from jax import lax
from jax.experimental import pallas as pl
from jax.experimental.pallas import tpu as pltpu
from jax.experimental.pallas import tpu_sc as plsc

DIM = 256
K = 8192
TILE_M = 512
WINDOWS = ((0, 2816), (2816, 2816), (5632, 2560))


def _argmin_body(x_ref, cb_ref, cf_ref, idx_ref, part_ref, c2_ref, iota_ref):
    m = pl.program_id(0)

    @pl.when(m == 0)
    def _init():
        c2_ref[...] = jnp.sum(cf_ref[...] * cf_ref[...], axis=0, keepdims=True)
        iota_ref[...] = lax.broadcasted_iota(jnp.int32, (1, K), 1).astype(
            jnp.float32
        )
        part_ref[...] = jnp.zeros((1, 1), jnp.float32)

    x = x_ref[...]
    xs = (-2.0 * x).astype(jnp.bfloat16)
    x2 = jnp.sum(x * x, axis=1, keepdims=True)

    acc_v = None
    for n, (off, w) in enumerate(WINDOWS):
        xc = jnp.dot(
            xs, cb_ref[:, off : off + w], preferred_element_type=jnp.float32
        )
        dist = (x2 + xc) + c2_ref[:, off : off + w]
        wv = jnp.min(dist, axis=1, keepdims=True)
        wi = jnp.min(
            jnp.where(dist == wv, iota_ref[:, off : off + w], jnp.float32(K)),
            axis=1,
            keepdims=True,
        ).astype(jnp.int32)
        if n == 0:
            acc_v, acc_i, acc_t = wv, wi, wv
        else:
            acc_up = acc_v.astype(jnp.bfloat16).astype(jnp.float32)
            take = wv < acc_up
            acc_v = jnp.where(take, wv, acc_up)
            acc_i = jnp.where(take, wi, acc_i)
            acc_t = jnp.where(take, wv, acc_t)

    idx_ref[...] = acc_i.reshape(1, 1, TILE_M)
    part_ref[...] += jnp.sum(acc_t, keepdims=True).reshape(1, 1)


def _argmin_call(xr, codebook, cb_bf16):
    n_tok = xr.shape[0]
    m_tiles = n_tok // TILE_M
    idx3, part = pl.pallas_call(
        _argmin_body,
        grid=(m_tiles,),
        in_specs=[
            pl.BlockSpec((TILE_M, DIM), lambda m: (m, 0)),
            pl.BlockSpec((DIM, K), lambda m: (0, 0)),
            pl.BlockSpec((DIM, K), lambda m: (0, 0)),
        ],
        out_specs=[
            pl.BlockSpec((1, 1, TILE_M), lambda m: (m, 0, 0)),
            pl.BlockSpec((1, 1), lambda m: (0, 0)),
        ],
        out_shape=[
            jax.ShapeDtypeStruct((m_tiles, 1, TILE_M), jnp.int32),
            jax.ShapeDtypeStruct((1, 1), jnp.float32),
        ],
        scratch_shapes=[
            pltpu.VMEM((1, K), jnp.float32),
            pltpu.VMEM((1, K), jnp.float32),
        ],
    )(xr, cb_bf16, codebook)
    return idx3.reshape(n_tok), part[0, 0]


def _make_gather(n_tok):
    info = plsc.get_sparse_core_info()
    nc, ns = info.num_cores, info.num_subcores
    nw = nc * ns
    b_per_w = n_tok // nw
    chunk = 128
    n_ch = b_per_w // chunk
    mesh = plsc.VectorSubcoreMesh(core_axis_name="c", subcore_axis_name="s")

    @functools.partial(
        pl.kernel,
        mesh=mesh,
        out_type=jax.ShapeDtypeStruct((n_tok, DIM), jnp.float32),
        scratch_types=[
            pltpu.VMEM((b_per_w,), jnp.int32),
            pltpu.VMEM((chunk, DIM), jnp.float32),
            pltpu.SemaphoreType.DMA,
        ],
    )
    def gather(table_hbm, idx_hbm, out_hbm, idx_v, rows_v, sem):
        wid = lax.axis_index("s") * nc + lax.axis_index("c")
        base = wid * b_per_w
        pltpu.sync_copy(idx_hbm.at[pl.ds(base, b_per_w)], idx_v)
        for ch in range(n_ch):
            pltpu.async_copy(
                table_hbm.at[idx_v.at[pl.ds(ch * chunk, chunk)]], rows_v, sem
            ).wait()
            pltpu.sync_copy(rows_v, out_hbm.at[pl.ds(base + ch * chunk, chunk)])

    return gather


def kernel(x_in, codebook):
    b, l, d = x_in.shape
    n_tok = b * l
    xr = x_in.reshape(n_tok, d)
    cb_bf16 = codebook.astype(jnp.bfloat16)
    idx, dist_sum = _argmin_call(xr, codebook, cb_bf16)
    table = codebook.T
    xq = _make_gather(n_tok)(table, idx)
    out = xq.reshape(b, l, d)
    loss = dist_sum / jnp.float32(n_tok * d)
    return out, idx.reshape(b, l), loss, loss
